# baseline (speedup 1.0000x reference)
import jax
import jax.numpy as jnp
from jax.experimental import pallas as pl
from jax.experimental.pallas import tpu as pltpu

_U = 256


def _gather_rows(table_ref, tok_ref, dst_ref, sem, n, chunk=8):
    def issue(c, _):
        base = c * chunk
        for j in range(chunk):
            i = base + j
            pltpu.make_async_copy(table_ref.at[tok_ref[i]],
                                  dst_ref.at[i], sem).start()
        return 0
    jax.lax.fori_loop(0, n // chunk, issue, 0)


def _wait_rows(table_ref, dst_ref, sem, n):
    pltpu.make_async_copy(table_ref.at[pl.ds(0, n)],
                          dst_ref.at[pl.ds(0, n)], sem).wait()


def _core_kernel(tok_e, tok_d, enc_embed, dec_embed,
                 enc_Wx, enc_Wh, enc_b, dec_Wxc, dec_Wxe, dec_b,
                 dec_W2Wh, W1, b12, Va_row,
                 h_out, enc_out, enc_proj, gx_buf, emb_buf,
                 sem_e, sem_d):
    S, B, U = enc_out.shape
    N = S * B

    def gru_gates(gx, gh, h):
        z = jax.nn.sigmoid(gx[:, :U] + gh[:, :U])
        r = jax.nn.sigmoid(gx[:, U:2 * U] + gh[:, U:2 * U])
        hh = jnp.tanh(gx[:, 2 * U:] + r * gh[:, 2 * U:])
        return z * h + (1.0 - z) * hh

    _gather_rows(enc_embed, tok_e, emb_buf, sem_e, N)
    _wait_rows(enc_embed, emb_buf, sem_e, N)
    gx_buf[...] = (jnp.dot(emb_buf[...], enc_Wx[...],
                           preferred_element_type=jnp.float32)
                   + enc_b[...]).reshape(S, B, 3 * U)

    _gather_rows(dec_embed, tok_d, emb_buf, sem_d, N)

    def enc_step(t, h):
        gh = jnp.dot(h, enc_Wh[...], preferred_element_type=jnp.float32)
        h_new = gru_gates(gx_buf[t], gh, h)
        enc_out[t] = h_new
        return h_new

    h_enc = jax.lax.fori_loop(0, S, enc_step, jnp.zeros((B, U), jnp.float32))

    eo = enc_out[...].reshape(N, U)
    enc_proj[...] = (jnp.dot(eo, W1[...], preferred_element_type=jnp.float32)
                     + b12[...]).reshape(S, B, U).astype(jnp.bfloat16)

    _wait_rows(dec_embed, emb_buf, sem_d, N)
    gx_buf[...] = (jnp.dot(emb_buf[...], dec_Wxe[...],
                           preferred_element_type=jnp.float32)
                   + dec_b[...]).reshape(S, B, 3 * U)

    def dec_step(t, h):
        hW = jnp.dot(h, dec_W2Wh[...], preferred_element_type=jnp.float32)
        dh = hW[:, :U].astype(jnp.bfloat16)
        gh = hW[:, U:]
        a = jnp.tanh(enc_proj[...] + dh[None, :, :])
        score = jnp.sum(a * Va_row[...][None], axis=-1)
        e = jnp.exp(score.astype(jnp.float32))
        rinv = 1.0 / jnp.sum(e, axis=0, keepdims=True)
        ctxu = jnp.sum(e[:, :, None] * enc_out[...], axis=0)
        ctx = ctxu * jnp.transpose(rinv)
        gx = (jnp.dot(ctx, dec_Wxc[...], preferred_element_type=jnp.float32)
              + gx_buf[t])
        h_new = gru_gates(gx, gh, h)
        h_out[t] = h_new.astype(jnp.bfloat16)
        return h_new

    jax.lax.fori_loop(0, S, dec_step, h_enc)


def _logits_kernel(h_ref, w_ref, b_ref, o_ref):
    S = h_ref.shape[0]
    U = _U
    nb = h_ref.shape[1] // U
    hcat = jnp.concatenate([h_ref[:, i * U:(i + 1) * U] for i in range(nb)],
                           axis=0)
    acc = jnp.dot(hcat, w_ref[...],
                  preferred_element_type=jnp.float32) + b_ref[...]
    o_ref[...] = acc.reshape(nb, S, acc.shape[-1])


def kernel(x, labels, enc_embed, enc_Wx, enc_Wh, enc_b,
           dec_embed, dec_Wx, dec_Wh, dec_b,
           W1, b1, W2, b2, Va, ba, Wfc, bfc):
    B, S = x.shape
    U = _U
    V = Wfc.shape[1]

    tok_e = jnp.transpose(x).reshape(-1)
    tok = jnp.concatenate([jnp.zeros((B, 1), labels.dtype),
                           labels[:, :-1]], axis=1)
    tok_d = jnp.transpose(tok).reshape(-1)

    h_seq = pl.pallas_call(
        _core_kernel,
        out_shape=jax.ShapeDtypeStruct((S, B, U), jnp.bfloat16),
        in_specs=[
            pl.BlockSpec(memory_space=pltpu.SMEM),
            pl.BlockSpec(memory_space=pltpu.SMEM),
            pl.BlockSpec(memory_space=pl.ANY),
            pl.BlockSpec(memory_space=pl.ANY),
        ] + [pl.BlockSpec()] * 10,
        scratch_shapes=[
            pltpu.VMEM((S, B, U), jnp.float32),
            pltpu.VMEM((S, B, U), jnp.bfloat16),
            pltpu.VMEM((S, B, 3 * U), jnp.float32),
            pltpu.VMEM((S * B, U), jnp.float32),
            pltpu.SemaphoreType.DMA,
            pltpu.SemaphoreType.DMA,
        ],
        compiler_params=pltpu.CompilerParams(
            vmem_limit_bytes=58 * 1024 * 1024,
        ),
        name="nmt_core",
    )(tok_e, tok_d, enc_embed, dec_embed,
      enc_Wx, enc_Wh, enc_b.reshape(1, 3 * U),
      dec_Wx[:U], dec_Wx[U:], dec_b.reshape(1, 3 * U),
      jnp.concatenate([W2, dec_Wh], axis=1),
      W1, (b1 + b2).reshape(1, U),
      Va.reshape(1, U).astype(jnp.bfloat16))

    h3 = h_seq.reshape(S, B * U)

    NB = 8
    BV = 3200
    nm = B // NB
    nv = V // BV
    logits = pl.pallas_call(
        _logits_kernel,
        out_shape=jax.ShapeDtypeStruct((B, S, V), jnp.float32),
        grid=(nv, nm),
        in_specs=[
            pl.BlockSpec((S, NB * U), lambda v, m: (0, m)),
            pl.BlockSpec((U, BV), lambda v, m: (0, v)),
            pl.BlockSpec((1, BV), lambda v, m: (0, v)),
        ],
        out_specs=pl.BlockSpec((NB, S, BV), lambda v, m: (m, 0, v)),
        compiler_params=pltpu.CompilerParams(
            dimension_semantics=("parallel", "arbitrary"),
            vmem_limit_bytes=48 * 1024 * 1024,
        ),
        name="nmt_logits",
    )(h3, Wfc.astype(jnp.bfloat16), bfc.reshape(1, V))

    return logits


# speedup vs baseline: 1.0363x; 1.0363x over previous
import jax
import jax.numpy as jnp
from jax.experimental import pallas as pl
from jax.experimental.pallas import tpu as pltpu

_U = 256


def _gather_rows(table_ref, tok_ref, dst_ref, sem, n, chunk=8):
    def issue(c, _):
        base = c * chunk
        for j in range(chunk):
            i = base + j
            pltpu.make_async_copy(table_ref.at[tok_ref[i]],
                                  dst_ref.at[i], sem).start()
        return 0
    jax.lax.fori_loop(0, n // chunk, issue, 0)


def _wait_rows(table_ref, dst_ref, sem, n):
    pltpu.make_async_copy(table_ref.at[pl.ds(0, n)],
                          dst_ref.at[pl.ds(0, n)], sem).wait()


def _core_kernel(tok_e, tok_d, enc_embed, dec_embed,
                 enc_Wx, enc_Wh, enc_b, dec_Wxc, dec_Wxe, dec_b,
                 dec_W2Wh, W1, b12, Va_row,
                 h_out, enc_out, enc_proj, gx_buf, emb_buf,
                 sem_e, sem_d):
    S, B, U = enc_out.shape
    N = S * B

    def gru_gates(gx, gh, h):
        z = jax.nn.sigmoid(gx[:, :U] + gh[:, :U])
        r = jax.nn.sigmoid(gx[:, U:2 * U] + gh[:, U:2 * U])
        hh = jnp.tanh(gx[:, 2 * U:] + r * gh[:, 2 * U:])
        return z * h + (1.0 - z) * hh

    _gather_rows(enc_embed, tok_e, emb_buf, sem_e, N)
    _wait_rows(enc_embed, emb_buf, sem_e, N)
    gx_buf[...] = (jnp.dot(emb_buf[...], enc_Wx[...],
                           preferred_element_type=jnp.float32)
                   + enc_b[...]).reshape(S, B, 3 * U)

    _gather_rows(dec_embed, tok_d, emb_buf, sem_d, N)

    def enc_step(t, h):
        gh = jnp.dot(h, enc_Wh[...], preferred_element_type=jnp.float32)
        h_new = gru_gates(gx_buf[t], gh, h)
        enc_out[t] = h_new
        return h_new

    h_enc = jax.lax.fori_loop(0, S, enc_step, jnp.zeros((B, U), jnp.float32))

    eo = enc_out[...].reshape(N, U)
    enc_proj[...] = (jnp.dot(eo, W1[...], preferred_element_type=jnp.float32)
                     + b12[...]).reshape(S, B, U)

    _wait_rows(dec_embed, emb_buf, sem_d, N)
    gx_buf[...] = (jnp.dot(emb_buf[...], dec_Wxe[...],
                           preferred_element_type=jnp.float32)
                   + dec_b[...]).reshape(S, B, 3 * U)

    def dec_step(t, h):
        hW = jnp.dot(h, dec_W2Wh[...], preferred_element_type=jnp.float32)
        dh = hW[:, :U]
        gh = hW[:, U:]
        a = jnp.tanh(enc_proj[...] + dh[None, :, :])
        score = jnp.sum(a * Va_row[...][None], axis=-1)
        e = jnp.exp(score)
        rinv = 1.0 / jnp.sum(e, axis=0, keepdims=True)
        ctxu = jnp.sum(e[:, :, None] * enc_out[...], axis=0)
        ctx = ctxu * jnp.transpose(rinv)
        gx = (jnp.dot(ctx, dec_Wxc[...], preferred_element_type=jnp.float32)
              + gx_buf[t])
        h_new = gru_gates(gx, gh, h)
        h_out[t] = h_new.astype(jnp.bfloat16)
        return h_new

    jax.lax.fori_loop(0, S, dec_step, h_enc)


def _logits_kernel(h_ref, w_ref, b_ref, o_ref):
    S = h_ref.shape[0]
    U = _U
    nb = h_ref.shape[1] // U
    hcat = jnp.concatenate([h_ref[:, i * U:(i + 1) * U] for i in range(nb)],
                           axis=0)
    acc = jnp.dot(hcat, w_ref[...],
                  preferred_element_type=jnp.float32) + b_ref[...]
    o_ref[...] = acc.reshape(nb, S, acc.shape[-1])


def kernel(x, labels, enc_embed, enc_Wx, enc_Wh, enc_b,
           dec_embed, dec_Wx, dec_Wh, dec_b,
           W1, b1, W2, b2, Va, ba, Wfc, bfc):
    B, S = x.shape
    U = _U
    V = Wfc.shape[1]

    tok_e = jnp.transpose(x).reshape(-1)
    tok = jnp.concatenate([jnp.zeros((B, 1), labels.dtype),
                           labels[:, :-1]], axis=1)
    tok_d = jnp.transpose(tok).reshape(-1)

    h_seq = pl.pallas_call(
        _core_kernel,
        out_shape=jax.ShapeDtypeStruct((S, B, U), jnp.bfloat16),
        in_specs=[
            pl.BlockSpec(memory_space=pltpu.SMEM),
            pl.BlockSpec(memory_space=pltpu.SMEM),
            pl.BlockSpec(memory_space=pl.ANY),
            pl.BlockSpec(memory_space=pl.ANY),
        ] + [pl.BlockSpec()] * 10,
        scratch_shapes=[
            pltpu.VMEM((S, B, U), jnp.float32),
            pltpu.VMEM((S, B, U), jnp.float32),
            pltpu.VMEM((S, B, 3 * U), jnp.float32),
            pltpu.VMEM((S * B, U), jnp.float32),
            pltpu.SemaphoreType.DMA,
            pltpu.SemaphoreType.DMA,
        ],
        compiler_params=pltpu.CompilerParams(
            vmem_limit_bytes=58 * 1024 * 1024,
        ),
        name="nmt_core",
    )(tok_e, tok_d, enc_embed, dec_embed,
      enc_Wx, enc_Wh, enc_b.reshape(1, 3 * U),
      dec_Wx[:U], dec_Wx[U:], dec_b.reshape(1, 3 * U),
      jnp.concatenate([W2, dec_Wh], axis=1),
      W1, (b1 + b2).reshape(1, U),
      Va.reshape(1, U))

    h3 = h_seq.reshape(S, B * U)

    NB = 8
    BV = 3200
    nm = B // NB
    nv = V // BV
    logits = pl.pallas_call(
        _logits_kernel,
        out_shape=jax.ShapeDtypeStruct((B, S, V), jnp.float32),
        grid=(nv, nm),
        in_specs=[
            pl.BlockSpec((S, NB * U), lambda v, m: (0, m)),
            pl.BlockSpec((U, BV), lambda v, m: (0, v)),
            pl.BlockSpec((1, BV), lambda v, m: (0, v)),
        ],
        out_specs=pl.BlockSpec((NB, S, BV), lambda v, m: (m, 0, v)),
        compiler_params=pltpu.CompilerParams(
            dimension_semantics=("parallel", "arbitrary"),
            vmem_limit_bytes=48 * 1024 * 1024,
        ),
        name="nmt_logits",
    )(h3, Wfc.astype(jnp.bfloat16), bfc.reshape(1, V))

    return logits


# speedup vs baseline: 1.0452x; 1.0086x over previous
import jax
import jax.numpy as jnp
from jax.experimental import pallas as pl
from jax.experimental.pallas import tpu as pltpu

_U = 256


def _gather_rows(table_ref, tok_ref, dst_ref, sem, n, chunk=8):
    def issue(c, _):
        base = c * chunk
        for j in range(chunk):
            i = base + j
            pltpu.make_async_copy(table_ref.at[tok_ref[i]],
                                  dst_ref.at[i], sem).start()
        return 0
    jax.lax.fori_loop(0, n // chunk, issue, 0)


def _wait_rows(table_ref, dst_ref, sem, n):
    pltpu.make_async_copy(table_ref.at[pl.ds(0, n)],
                          dst_ref.at[pl.ds(0, n)], sem).wait()


def _core_kernel(tok_e, tok_d, enc_embed, dec_embed,
                 enc_Wx, enc_Wh, enc_b, dec_Wxc, dec_Wxe, dec_b,
                 dec_W2Wh, W1, b12, Va_row,
                 h_out, enc_out, enc_proj, gx_buf, emb_buf,
                 sem_e, sem_d):
    S, B, U = enc_out.shape
    N = S * B

    def gru_gates(gx, gh, h):
        z = jax.nn.sigmoid(gx[:, :U] + gh[:, :U])
        r = jax.nn.sigmoid(gx[:, U:2 * U] + gh[:, U:2 * U])
        hh = jnp.tanh(gx[:, 2 * U:] + r * gh[:, 2 * U:])
        return z * h + (1.0 - z) * hh

    _gather_rows(enc_embed, tok_e, emb_buf, sem_e, N)
    _wait_rows(enc_embed, emb_buf, sem_e, N)
    gx_buf[...] = (jnp.dot(emb_buf[...], enc_Wx[...],
                           preferred_element_type=jnp.float32)
                   + enc_b[...]).reshape(S, B, 3 * U)

    _gather_rows(dec_embed, tok_d, emb_buf, sem_d, N)

    def enc_step(t, h):
        gh = jnp.dot(h, enc_Wh[...], preferred_element_type=jnp.float32)
        h_new = gru_gates(gx_buf[t], gh, h)
        enc_out[t] = h_new
        return h_new

    h_enc = jax.lax.fori_loop(0, S, enc_step, jnp.zeros((B, U), jnp.float32))

    eo = enc_out[...].reshape(N, U)
    enc_proj[...] = (jnp.dot(eo, W1[...], preferred_element_type=jnp.float32)
                     + b12[...]).reshape(S, B, U)

    _wait_rows(dec_embed, emb_buf, sem_d, N)
    gx_buf[...] = (jnp.dot(emb_buf[...], dec_Wxe[...],
                           preferred_element_type=jnp.float32)
                   + dec_b[...]).reshape(S, B, 3 * U)

    def dec_step(t, h):
        hW = jnp.dot(h, dec_W2Wh[...], preferred_element_type=jnp.float32)
        dh = hW[:, :U]
        gh = hW[:, U:]
        CH = 32
        se = jnp.zeros((1, B), jnp.float32)
        ctxu = jnp.zeros((B, U), jnp.float32)
        for c in range(S // CH):
            sl = slice(c * CH, (c + 1) * CH)
            a_c = jnp.tanh(enc_proj[sl] + dh[None, :, :])
            sc_c = jnp.sum(a_c * Va_row[...][None], axis=-1)
            e_c = jnp.exp(sc_c)
            se = se + jnp.sum(e_c, axis=0, keepdims=True)
            ctxu = ctxu + jnp.sum(e_c[:, :, None] * enc_out[sl], axis=0)
        ctx = ctxu * jnp.transpose(1.0 / se)
        gx = (jnp.dot(ctx, dec_Wxc[...], preferred_element_type=jnp.float32)
              + gx_buf[t])
        h_new = gru_gates(gx, gh, h)
        h_out[t] = h_new.astype(jnp.bfloat16)
        return h_new

    jax.lax.fori_loop(0, S, dec_step, h_enc)


def _logits_kernel(h_ref, w_ref, b_ref, o_ref):
    S = h_ref.shape[0]
    U = _U
    nb = h_ref.shape[1] // U
    hcat = jnp.concatenate([h_ref[:, i * U:(i + 1) * U] for i in range(nb)],
                           axis=0)
    acc = jnp.dot(hcat, w_ref[...],
                  preferred_element_type=jnp.float32) + b_ref[...]
    o_ref[...] = acc.reshape(nb, S, acc.shape[-1])


def kernel(x, labels, enc_embed, enc_Wx, enc_Wh, enc_b,
           dec_embed, dec_Wx, dec_Wh, dec_b,
           W1, b1, W2, b2, Va, ba, Wfc, bfc):
    B, S = x.shape
    U = _U
    V = Wfc.shape[1]

    tok_e = jnp.transpose(x).reshape(-1)
    tok = jnp.concatenate([jnp.zeros((B, 1), labels.dtype),
                           labels[:, :-1]], axis=1)
    tok_d = jnp.transpose(tok).reshape(-1)

    h_seq = pl.pallas_call(
        _core_kernel,
        out_shape=jax.ShapeDtypeStruct((S, B, U), jnp.bfloat16),
        in_specs=[
            pl.BlockSpec(memory_space=pltpu.SMEM),
            pl.BlockSpec(memory_space=pltpu.SMEM),
            pl.BlockSpec(memory_space=pl.ANY),
            pl.BlockSpec(memory_space=pl.ANY),
        ] + [pl.BlockSpec()] * 10,
        scratch_shapes=[
            pltpu.VMEM((S, B, U), jnp.float32),
            pltpu.VMEM((S, B, U), jnp.float32),
            pltpu.VMEM((S, B, 3 * U), jnp.float32),
            pltpu.VMEM((S * B, U), jnp.float32),
            pltpu.SemaphoreType.DMA,
            pltpu.SemaphoreType.DMA,
        ],
        compiler_params=pltpu.CompilerParams(
            vmem_limit_bytes=58 * 1024 * 1024,
        ),
        name="nmt_core",
    )(tok_e, tok_d, enc_embed, dec_embed,
      enc_Wx, enc_Wh, enc_b.reshape(1, 3 * U),
      dec_Wx[:U], dec_Wx[U:], dec_b.reshape(1, 3 * U),
      jnp.concatenate([W2, dec_Wh], axis=1),
      W1, (b1 + b2).reshape(1, U),
      Va.reshape(1, U))

    h3 = h_seq.reshape(S, B * U)

    NB = 8
    BV = 3200
    nm = B // NB
    nv = V // BV
    logits = pl.pallas_call(
        _logits_kernel,
        out_shape=jax.ShapeDtypeStruct((B, S, V), jnp.float32),
        grid=(nv, nm),
        in_specs=[
            pl.BlockSpec((S, NB * U), lambda v, m: (0, m)),
            pl.BlockSpec((U, BV), lambda v, m: (0, v)),
            pl.BlockSpec((1, BV), lambda v, m: (0, v)),
        ],
        out_specs=pl.BlockSpec((NB, S, BV), lambda v, m: (m, 0, v)),
        compiler_params=pltpu.CompilerParams(
            dimension_semantics=("parallel", "arbitrary"),
            vmem_limit_bytes=48 * 1024 * 1024,
        ),
        name="nmt_logits",
    )(h3, Wfc.astype(jnp.bfloat16), bfc.reshape(1, V))

    return logits
